# Initial kernel scaffold; baseline (speedup 1.0000x reference)
import jax
import jax.numpy as jnp
from jax.experimental import pallas as pl


def kernel(inp, pref, suffixes, chrs, input_table, prefix_table, suffix_table):
    raise NotImplementedError("write your pallas kernel here")


# speedup vs baseline: 8.5329x; 8.5329x over previous
import functools

import jax
import jax.numpy as jnp
from jax import lax
from jax.experimental import pallas as pl
from jax.experimental.pallas import tpu as pltpu
from jax.experimental.pallas import tpu_sc as plsc

LANES = 16
NW = 32
CHUNK = 128


@functools.lru_cache(maxsize=None)
def _build(B, D):
    assert B % (NW * CHUNK) == 0
    bpw = B // NW
    nchunk = bpw // CHUNK
    mesh = plsc.VectorSubcoreMesh(core_axis_name="c", subcore_axis_name="s")

    @functools.partial(
        pl.kernel,
        mesh=mesh,
        compiler_params=pltpu.CompilerParams(use_tc_tiling_on_sc=False),
        out_type=jax.ShapeDtypeStruct((B, D), jnp.float32),
        scratch_types=[
            pltpu.VMEM((bpw,), jnp.int32),
            pltpu.VMEM((bpw,), jnp.int32),
            pltpu.VMEM((bpw,), jnp.int32),
            pltpu.VMEM((CHUNK, D), jnp.float32),
            pltpu.VMEM((CHUNK, D), jnp.float32),
            pltpu.VMEM((CHUNK, D), jnp.float32),
            pltpu.SemaphoreType.DMA,
        ],
    )
    def embed(eidx_hbm, pidx_hbm, sidx_hbm, etab, ptab, stab, out,
              eidx, pidx, sidx, ebuf, pbuf, sbuf, sem):
        wid = lax.axis_index("s") * 2 + lax.axis_index("c")
        base = wid * bpw
        pltpu.sync_copy(eidx_hbm.at[pl.ds(base, bpw)], eidx)
        pltpu.sync_copy(pidx_hbm.at[pl.ds(base, bpw)], pidx)
        pltpu.sync_copy(sidx_hbm.at[pl.ds(base, bpw)], sidx)

        def chunk_body(j, carry):
            off = j * CHUNK
            ce = pltpu.async_copy(etab.at[eidx.at[pl.ds(off, CHUNK)]], ebuf, sem)
            cp = pltpu.async_copy(ptab.at[pidx.at[pl.ds(off, CHUNK)]], pbuf, sem)
            cs = pltpu.async_copy(stab.at[sidx.at[pl.ds(off, CHUNK)]], sbuf, sem)
            ce.wait()
            cp.wait()
            cs.wait()

            def row_body(r, c):
                for k in range(D // LANES):
                    sl = pl.ds(k * LANES, LANES)
                    ebuf[r, sl] = ebuf[r, sl] + pbuf[r, sl] + sbuf[r, sl]
                return c

            lax.fori_loop(0, CHUNK, row_body, 0)
            pltpu.sync_copy(ebuf, out.at[pl.ds(base + off, CHUNK)])
            return carry

        lax.fori_loop(0, nchunk, chunk_body, 0)

    return embed


def kernel(inp, pref, suffixes, chrs, input_table, prefix_table, suffix_table):
    batch, seq = inp.shape
    D = input_table.shape[1]
    B = batch * seq
    e = inp.reshape(B).astype(jnp.int32)
    p = pref.reshape(B).astype(jnp.int32)
    s = suffixes.reshape(B).astype(jnp.int32)
    out = _build(B, D)(e, p, s, input_table, prefix_table, suffix_table)
    return out.reshape(batch, seq, D)


# speedup vs baseline: 8.6912x; 1.0185x over previous
import functools

import jax
import jax.numpy as jnp
from jax import lax
from jax.experimental import pallas as pl
from jax.experimental.pallas import tpu as pltpu
from jax.experimental.pallas import tpu_sc as plsc

LANES = 16
NW = 32
CHUNK = 128


@functools.lru_cache(maxsize=None)
def _build(B, D):
    assert B % (NW * CHUNK) == 0
    bpw = B // NW
    nchunk = bpw // CHUNK
    mesh = plsc.VectorSubcoreMesh(core_axis_name="c", subcore_axis_name="s")

    @functools.partial(
        pl.kernel,
        mesh=mesh,
        compiler_params=pltpu.CompilerParams(use_tc_tiling_on_sc=False),
        out_type=jax.ShapeDtypeStruct((B, D), jnp.float32),
        scratch_types=[
            pltpu.VMEM((bpw,), jnp.int32),
            pltpu.VMEM((bpw,), jnp.int32),
            pltpu.VMEM((bpw,), jnp.int32),
            pltpu.VMEM((CHUNK, D), jnp.float32),
            pltpu.VMEM((CHUNK, D), jnp.float32),
            pltpu.VMEM((CHUNK, D), jnp.float32),
            pltpu.SemaphoreType.DMA,
        ],
    )
    def embed(eidx_hbm, pidx_hbm, sidx_hbm, etab, ptab, stab, out,
              eidx, pidx, sidx, ebuf, pbuf, sbuf, sem):
        wid = lax.axis_index("s") * 2 + lax.axis_index("c")
        base = wid * bpw
        pltpu.sync_copy(eidx_hbm.at[pl.ds(base, bpw)], eidx)
        pltpu.sync_copy(pidx_hbm.at[pl.ds(base, bpw)], pidx)
        pltpu.sync_copy(sidx_hbm.at[pl.ds(base, bpw)], sidx)

        def chunk_body(j, carry):
            off = j * CHUNK
            ce = pltpu.async_copy(etab.at[eidx.at[pl.ds(off, CHUNK)]], ebuf, sem)
            ce.wait()
            cp = pltpu.async_copy(ptab.at[pidx.at[pl.ds(off, CHUNK)]], ebuf, sem,
                                  add=True)
            cs = pltpu.async_copy(stab.at[sidx.at[pl.ds(off, CHUNK)]], ebuf, sem,
                                  add=True)
            cp.wait()
            cs.wait()
            pltpu.sync_copy(ebuf, out.at[pl.ds(base + off, CHUNK)])
            return carry

        lax.fori_loop(0, nchunk, chunk_body, 0)

    return embed


def kernel(inp, pref, suffixes, chrs, input_table, prefix_table, suffix_table):
    batch, seq = inp.shape
    D = input_table.shape[1]
    B = batch * seq
    e = inp.reshape(B).astype(jnp.int32)
    p = pref.reshape(B).astype(jnp.int32)
    s = suffixes.reshape(B).astype(jnp.int32)
    out = _build(B, D)(e, p, s, input_table, prefix_table, suffix_table)
    return out.reshape(batch, seq, D)


# speedup vs baseline: 10.5187x; 1.2103x over previous
import functools

import jax
import jax.numpy as jnp
from jax import lax
from jax.experimental import pallas as pl
from jax.experimental.pallas import tpu as pltpu
from jax.experimental.pallas import tpu_sc as plsc

LANES = 16
NW = 32
CHUNK = 128


NBUF = 10


@functools.lru_cache(maxsize=None)
def _build(B, D):
    assert B % (NW * CHUNK * NBUF) == 0
    bpw = B // NW
    ngroup = bpw // (CHUNK * NBUF)
    mesh = plsc.VectorSubcoreMesh(core_axis_name="c", subcore_axis_name="s")

    @functools.partial(
        pl.kernel,
        mesh=mesh,
        compiler_params=pltpu.CompilerParams(use_tc_tiling_on_sc=False),
        out_type=jax.ShapeDtypeStruct((B, D), jnp.float32),
        scratch_types=[
            pltpu.VMEM((bpw,), jnp.int32),
            pltpu.VMEM((bpw,), jnp.int32),
            pltpu.VMEM((bpw,), jnp.int32),
            pltpu.VMEM((NBUF, CHUNK, D), jnp.float32),
            pltpu.SemaphoreType.DMA((NBUF,)),
            pltpu.SemaphoreType.DMA((NBUF,)),
            pltpu.SemaphoreType.DMA((NBUF,)),
        ],
    )
    def embed(eidx_hbm, pidx_hbm, sidx_hbm, etab, ptab, stab, out,
              eidx, pidx, sidx, buf, sem_e, sem_a, sem_o):
        wid = lax.axis_index("s") * 2 + lax.axis_index("c")
        base = wid * bpw
        pltpu.sync_copy(eidx_hbm.at[pl.ds(base, bpw)], eidx)
        pltpu.sync_copy(pidx_hbm.at[pl.ds(base, bpw)], pidx)
        pltpu.sync_copy(sidx_hbm.at[pl.ds(base, bpw)], sidx)

        def group_body(g, carry):
            goff = g * (NBUF * CHUNK)
            ce = []
            for b in range(NBUF):
                off = goff + b * CHUNK
                ce.append(pltpu.async_copy(
                    etab.at[eidx.at[pl.ds(off, CHUNK)]], buf.at[b],
                    sem_e.at[b]))
            ca = []
            for b in range(NBUF):
                off = goff + b * CHUNK
                ce[b].wait()
                ca.append(pltpu.async_copy(
                    ptab.at[pidx.at[pl.ds(off, CHUNK)]], buf.at[b],
                    sem_a.at[b], add=True))
                ca.append(pltpu.async_copy(
                    stab.at[sidx.at[pl.ds(off, CHUNK)]], buf.at[b],
                    sem_a.at[b], add=True))
            co = []
            for b in range(NBUF):
                off = goff + b * CHUNK
                ca[2 * b].wait()
                ca[2 * b + 1].wait()
                co.append(pltpu.async_copy(
                    buf.at[b], out.at[pl.ds(base + off, CHUNK)], sem_o.at[b]))
            for b in range(NBUF):
                co[b].wait()
            return carry

        lax.fori_loop(0, ngroup, group_body, 0)

    return embed


def kernel(inp, pref, suffixes, chrs, input_table, prefix_table, suffix_table):
    batch, seq = inp.shape
    D = input_table.shape[1]
    B = batch * seq
    e = inp.reshape(B).astype(jnp.int32)
    p = pref.reshape(B).astype(jnp.int32)
    s = suffixes.reshape(B).astype(jnp.int32)
    out = _build(B, D)(e, p, s, input_table, prefix_table, suffix_table)
    return out.reshape(batch, seq, D)
